# Initial kernel scaffold; baseline (speedup 1.0000x reference)
import jax
import jax.numpy as jnp
from jax.experimental import pallas as pl


def kernel(x, edge_index, edge_attr, W_l, b_l, W_r, b_r, W_e, att, bias, W1, b1, W2, b2):
    raise NotImplementedError("write your pallas kernel here")


# speedup vs baseline: 13.8843x; 13.8843x over previous
import functools

import jax
import jax.numpy as jnp
from jax import lax
from jax.experimental import pallas as pl
from jax.experimental.pallas import tpu as pltpu
from jax.experimental.pallas import tpu_sc as plsc

_N = 100000
_E = 6400000
_C = 6
_OUT = 3

_K = 512
_SUB = _K // 128
_NW = 32
_PT = 200704
_EP = _PT * _NW
_NCH = _PT // _K
_ACC_N = 100016
_AW = 16
_ZR = _ACC_N // 16
_WR = _N // 16
_BLK = 2000


def _prep_body(x_ref, wl_ref, bl_ref, wr_ref, br_ref, xl_ref, xr_ref):
    x = x_ref[...]
    xl_ref[...] = x @ wl_ref[...] + bl_ref[...]
    xr_ref[...] = x @ wr_ref[...] + br_ref[...]


def _prep(x, wl8, bl8, wr8, br8):
    grid = (_N // _BLK,)
    return pl.pallas_call(
        _prep_body,
        grid=grid,
        in_specs=[
            pl.BlockSpec((_BLK, _C), lambda i: (i, 0)),
            pl.BlockSpec((_C, 8), lambda i: (0, 0)),
            pl.BlockSpec((1, 8), lambda i: (0, 0)),
            pl.BlockSpec((_C, 8), lambda i: (0, 0)),
            pl.BlockSpec((1, 8), lambda i: (0, 0)),
        ],
        out_specs=[
            pl.BlockSpec((_BLK, 8), lambda i: (i, 0)),
            pl.BlockSpec((_BLK, 8), lambda i: (i, 0)),
        ],
        out_shape=[
            jax.ShapeDtypeStruct((_N, 8), jnp.float32),
            jax.ShapeDtypeStruct((_N, 8), jnp.float32),
        ],
    )(x, wl8, bl8, wr8, br8)


def _edge_body(src_hbm, dst_hbm, attr_hbm, xl_hbm, xr_hbm, zero_hbm, par_hbm,
               out_hbm, acc, src_v, dst_v, attr_v, xl_v, xr_v, s_v, par_v,
               sem):
    cid = lax.axis_index("c")
    sid = lax.axis_index("s")
    wid = sid * 2 + cid

    pltpu.sync_copy(par_hbm, par_v)
    pltpu.sync_copy(zero_hbm.at[pl.ds(0, _K)], s_v)
    pltpu.sync_copy(zero_hbm.at[pl.ds(sid * _ZR, _ZR)],
                    acc.at[pl.ds(sid * _ZR, _ZR)])
    plsc.subcore_barrier()

    lanes = lax.iota(jnp.int32, 16)
    cols = [jnp.full((16,), j, jnp.int32) for j in range(10)]
    p_lo = par_v[pl.ds(0, 16)]
    p_hi = par_v[pl.ds(8, 16)]
    we0 = [p_lo[j] for j in range(6)]
    we1 = [p_lo[6 + j] for j in range(6)]
    att_s = [p_hi[4 + j] for j in range(6)]
    ones16 = jnp.ones((16,), jnp.float32)

    def chunk_body(i, carry):
        e_base = wid * _PT + i * _K
        r_base = wid * (_PT // 128) + i * _SUB
        c1 = pltpu.async_copy(src_hbm.at[pl.ds(r_base, _SUB)], src_v, sem)
        c2 = pltpu.async_copy(dst_hbm.at[pl.ds(r_base, _SUB)], dst_v, sem)
        c3 = pltpu.async_copy(attr_hbm.at[pl.ds(e_base, _K)], attr_v, sem)
        c1.wait()
        c2.wait()
        c3.wait()
        gs = []
        for c in range(_SUB):
            gs.append(pltpu.async_copy(xl_hbm.at[src_v.at[c]],
                                       xl_v.at[pl.ds(c * 128, 128)], sem))
            gs.append(pltpu.async_copy(xr_hbm.at[dst_v.at[c]],
                                       xr_v.at[pl.ds(c * 128, 128)], sem))
        for g in gs:
            g.wait()

        def group_body(g, c2_):
            eidx = g * 16 + lanes
            a0 = plsc.load_gather(attr_v, [eidx, cols[0]])
            a1 = plsc.load_gather(attr_v, [eidx, cols[1]])
            xl = [plsc.load_gather(xl_v, [eidx, cols[j]]) for j in range(6)]
            xr = [plsc.load_gather(xr_v, [eidx, cols[j]]) for j in range(6)]
            logit = jnp.zeros((16,), jnp.float32)
            for j in range(6):
                m = xl[j] + xr[j] + a0 * we0[j] + a1 * we1[j]
                lk = jnp.maximum(m, m * 0.2)
                logit = logit + lk * att_s[j]
            ex = jnp.exp(logit)
            plsc.store_scatter(s_v, [eidx, cols[0]], ex)
            for j in range(6):
                plsc.store_scatter(s_v, [eidx, cols[1 + j]], ex * xl[j])
            plsc.store_scatter(s_v, [eidx, cols[7]], ones16)
            plsc.store_scatter(s_v, [eidx, cols[8]], a0)
            plsc.store_scatter(s_v, [eidx, cols[9]], a1)
            return c2_

        lax.fori_loop(0, _K // 16, group_body, 0)
        ss = []
        for c in range(_SUB):
            ss.append(pltpu.async_copy(s_v.at[pl.ds(c * 128, 128)],
                                       acc.at[dst_v.at[c]], sem, add=True))
        for s in ss:
            s.wait()
        return carry

    lax.fori_loop(0, _NCH, chunk_body, 0)
    plsc.subcore_barrier()
    pltpu.sync_copy(acc.at[pl.ds(sid * _WR, _WR)],
                    out_hbm.at[cid, pl.ds(sid * _WR, _WR)])


_edge_kernel = functools.partial(
    pl.kernel,
    out_type=jax.ShapeDtypeStruct((2, _N, _AW), jnp.float32),
    mesh=plsc.VectorSubcoreMesh(core_axis_name="c", subcore_axis_name="s"),
    compiler_params=pltpu.CompilerParams(use_tc_tiling_on_sc=False,
                                         needs_layout_passes=False),
    scratch_types=[
        pltpu.VMEM_SHARED((_ACC_N, _AW), jnp.float32),
        pltpu.VMEM((_SUB, 128), jnp.int32),
        pltpu.VMEM((_SUB, 128), jnp.int32),
        pltpu.VMEM((_K, 2), jnp.float32),
        pltpu.VMEM((_K, 8), jnp.float32),
        pltpu.VMEM((_K, 8), jnp.float32),
        pltpu.VMEM((_K, _AW), jnp.float32),
        pltpu.VMEM((24,), jnp.float32),
        pltpu.SemaphoreType.DMA,
    ],
)(_edge_body)


def _finish_body(acc_ref, xl_ref, xr_ref, we_ref, att_ref, bias_ref,
                 w1_ref, b1_ref, w2_ref, b2_ref, out_ref):
    acc = acc_ref[0] + acc_ref[1]
    den_e = acc[:, 0]
    num = acc[:, 1:7]
    deg = acc[:, 7]
    asum = acc[:, 8:10]
    sl = asum / jnp.maximum(deg, 1.0)[:, None]
    xl = xl_ref[:, :6]
    xr = xr_ref[:, :6]
    msg = xl + xr + sl @ we_ref[...]
    lk = jnp.maximum(msg, 0.2 * msg)
    logit = jnp.sum(lk * att_ref[...], axis=1)
    ex = jnp.exp(logit)
    den = den_e + ex
    numt = num + ex[:, None] * xl
    out0 = numt / (den[:, None] + 1e-16) + bias_ref[...]
    h = jnp.maximum(out0, 0.0)
    h = jnp.maximum(h @ w1_ref[...] + b1_ref[...], 0.0)
    h = jnp.maximum(h @ w2_ref[...] + b2_ref[...], 0.0)
    out_ref[...] = h * 2.0 - 1.0


def _finish(acc2, xl_pad, xr_pad, we, att2, bias2, w1, b12, w2, b22):
    grid = (_N // _BLK,)
    return pl.pallas_call(
        _finish_body,
        grid=grid,
        in_specs=[
            pl.BlockSpec((2, _BLK, _AW), lambda i: (0, i, 0)),
            pl.BlockSpec((_BLK, 8), lambda i: (i, 0)),
            pl.BlockSpec((_BLK, 8), lambda i: (i, 0)),
            pl.BlockSpec((2, _C), lambda i: (0, 0)),
            pl.BlockSpec((1, _C), lambda i: (0, 0)),
            pl.BlockSpec((1, _C), lambda i: (0, 0)),
            pl.BlockSpec((_C, _C), lambda i: (0, 0)),
            pl.BlockSpec((1, _C), lambda i: (0, 0)),
            pl.BlockSpec((_C, _OUT), lambda i: (0, 0)),
            pl.BlockSpec((1, _OUT), lambda i: (0, 0)),
        ],
        out_specs=pl.BlockSpec((_BLK, _OUT), lambda i: (i, 0)),
        out_shape=jax.ShapeDtypeStruct((_N, _OUT), jnp.float32),
    )(acc2, xl_pad, xr_pad, we, att2, bias2, w1, b12, w2, b22)


def kernel(x, edge_index, edge_attr, W_l, b_l, W_r, b_r, W_e, att, bias,
           W1, b1, W2, b2):
    wl8 = jnp.pad(W_l, ((0, 0), (0, 2)))
    bl8 = jnp.pad(b_l, (0, 2)).reshape(1, 8)
    wr8 = jnp.pad(W_r, ((0, 0), (0, 2)))
    br8 = jnp.pad(b_r, (0, 2)).reshape(1, 8)
    xl_pad, xr_pad = _prep(x, wl8, bl8, wr8, br8)

    pad = _EP - _E
    src_p = jnp.concatenate(
        [edge_index[0], jnp.zeros((pad,), jnp.int32)]).reshape(_EP // 128, 128)
    dst_p = jnp.concatenate(
        [edge_index[1], jnp.full((pad,), _N, jnp.int32)]).reshape(_EP // 128, 128)
    attr_p = jnp.concatenate(
        [edge_attr, jnp.zeros((pad, 2), jnp.float32)], axis=0)
    params = jnp.concatenate(
        [W_e.reshape(-1), att, jnp.zeros((6,), jnp.float32)])
    zeros_acc = jnp.zeros((_ACC_N, _AW), jnp.float32)

    acc2 = _edge_kernel(src_p, dst_p, attr_p, xl_pad, xr_pad, zeros_acc,
                        params)

    return _finish(acc2, xl_pad, xr_pad, W_e, att.reshape(1, _C),
                   bias.reshape(1, _C), W1, b1.reshape(1, _C), W2,
                   b2.reshape(1, _OUT))


# speedup vs baseline: 76.3086x; 5.4960x over previous
import functools

import jax
import jax.numpy as jnp
from jax import lax
from jax.experimental import pallas as pl
from jax.experimental.pallas import tpu as pltpu
from jax.experimental.pallas import tpu_sc as plsc

_N = 100000
_E = 6400000
_C = 6
_OUT = 3

_K = 512
_SUB = _K // 128
_NW = 32
_PT = 200704
_EP = _PT * _NW
_NCH = _PT // _K
_ACC_N = 100016
_AW = 16
_ZR = _ACC_N // 16
_WR = _N // 16
_BLK = 2000


def _prep_body(x_ref, wl_ref, bl_ref, wr_ref, br_ref, xl_ref, xr_ref):
    x = x_ref[...]
    xl_ref[...] = x @ wl_ref[...] + bl_ref[...]
    xr_ref[...] = x @ wr_ref[...] + br_ref[...]


def _prep(x, wl8, bl8, wr8, br8):
    grid = (_N // _BLK,)
    return pl.pallas_call(
        _prep_body,
        grid=grid,
        in_specs=[
            pl.BlockSpec((_BLK, _C), lambda i: (i, 0)),
            pl.BlockSpec((_C, 8), lambda i: (0, 0)),
            pl.BlockSpec((1, 8), lambda i: (0, 0)),
            pl.BlockSpec((_C, 8), lambda i: (0, 0)),
            pl.BlockSpec((1, 8), lambda i: (0, 0)),
        ],
        out_specs=[
            pl.BlockSpec((_BLK, 8), lambda i: (i, 0)),
            pl.BlockSpec((_BLK, 8), lambda i: (i, 0)),
        ],
        out_shape=[
            jax.ShapeDtypeStruct((_N, 8), jnp.float32),
            jax.ShapeDtypeStruct((_N, 8), jnp.float32),
        ],
    )(x, wl8, bl8, wr8, br8)


def _edge_body(src_hbm, dst_hbm, a0_hbm, a1_hbm, xl_hbm, xr_hbm, zero_hbm,
               par_hbm, out_hbm, acc, src_v, dst_v, a0_v, a1_v, xl_v, xr_v,
               s_v, par_v, sem):
    cid = lax.axis_index("c")
    sid = lax.axis_index("s")
    wid = sid * 2 + cid

    pltpu.sync_copy(par_hbm, par_v)
    pltpu.sync_copy(zero_hbm.at[pl.ds(0, _K)], s_v)
    pltpu.sync_copy(zero_hbm.at[pl.ds(sid * _ZR, _ZR)],
                    acc.at[pl.ds(sid * _ZR, _ZR)])
    plsc.subcore_barrier()

    lanes = lax.iota(jnp.int32, 16)
    cols = [jnp.full((16,), j, jnp.int32) for j in range(10)]
    p_lo = par_v[pl.ds(0, 16)]
    p_hi = par_v[pl.ds(8, 16)]
    we0 = [p_lo[j] for j in range(6)]
    we1 = [p_lo[6 + j] for j in range(6)]
    att_s = [p_hi[4 + j] for j in range(6)]
    ones16 = jnp.ones((16,), jnp.float32)

    def chunk_body(i, carry):
        e_base = wid * _PT + i * _K
        r_base = wid * (_PT // 128) + i * _SUB
        c1 = pltpu.async_copy(src_hbm.at[pl.ds(r_base, _SUB)], src_v, sem)
        c2 = pltpu.async_copy(dst_hbm.at[pl.ds(r_base, _SUB)], dst_v, sem)
        c3 = pltpu.async_copy(a0_hbm.at[pl.ds(e_base, _K)], a0_v, sem)
        c4 = pltpu.async_copy(a1_hbm.at[pl.ds(e_base, _K)], a1_v, sem)
        c1.wait()
        c2.wait()
        c3.wait()
        c4.wait()
        gs = []
        for c in range(_SUB):
            gs.append(pltpu.async_copy(xl_hbm.at[src_v.at[c]],
                                       xl_v.at[pl.ds(c * 128, 128)], sem))
            gs.append(pltpu.async_copy(xr_hbm.at[dst_v.at[c]],
                                       xr_v.at[pl.ds(c * 128, 128)], sem))
        for g in gs:
            g.wait()

        def group_body(g, c2_):
            eidx = g * 16 + lanes
            a0 = a0_v[pl.ds(g * 16, 16)]
            a1 = a1_v[pl.ds(g * 16, 16)]
            xl = [plsc.load_gather(xl_v, [eidx, cols[j]]) for j in range(6)]
            xr = [plsc.load_gather(xr_v, [eidx, cols[j]]) for j in range(6)]
            logit = jnp.zeros((16,), jnp.float32)
            for j in range(6):
                m = xl[j] + xr[j] + a0 * we0[j] + a1 * we1[j]
                lk = jnp.maximum(m, m * 0.2)
                logit = logit + lk * att_s[j]
            ex = jnp.exp(logit)
            plsc.store_scatter(s_v, [eidx, cols[0]], ex)
            for j in range(6):
                plsc.store_scatter(s_v, [eidx, cols[1 + j]], ex * xl[j])
            plsc.store_scatter(s_v, [eidx, cols[7]], ones16)
            plsc.store_scatter(s_v, [eidx, cols[8]], a0)
            plsc.store_scatter(s_v, [eidx, cols[9]], a1)
            return c2_

        lax.fori_loop(0, _K // 16, group_body, 0)
        ss = []
        for c in range(_SUB):
            ss.append(pltpu.async_copy(s_v.at[pl.ds(c * 128, 128)],
                                       acc.at[dst_v.at[c]], sem, add=True))
        for s in ss:
            s.wait()
        return carry

    lax.fori_loop(0, _NCH, chunk_body, 0)
    plsc.subcore_barrier()
    pltpu.sync_copy(acc.at[pl.ds(sid * _WR, _WR)],
                    out_hbm.at[cid, pl.ds(sid * _WR, _WR)])


_edge_kernel = functools.partial(
    pl.kernel,
    out_type=jax.ShapeDtypeStruct((2, _N, _AW), jnp.float32),
    mesh=plsc.VectorSubcoreMesh(core_axis_name="c", subcore_axis_name="s"),
    compiler_params=pltpu.CompilerParams(use_tc_tiling_on_sc=False,
                                         needs_layout_passes=False),
    scratch_types=[
        pltpu.VMEM_SHARED((_ACC_N, _AW), jnp.float32),
        pltpu.VMEM((_SUB, 128), jnp.int32),
        pltpu.VMEM((_SUB, 128), jnp.int32),
        pltpu.VMEM((_K,), jnp.float32),
        pltpu.VMEM((_K,), jnp.float32),
        pltpu.VMEM((_K, 8), jnp.float32),
        pltpu.VMEM((_K, 8), jnp.float32),
        pltpu.VMEM((_K, _AW), jnp.float32),
        pltpu.VMEM((24,), jnp.float32),
        pltpu.SemaphoreType.DMA,
    ],
)(_edge_body)


def _finish_body(acc_ref, xl_ref, xr_ref, we_ref, att_ref, bias_ref,
                 w1_ref, b1_ref, w2_ref, b2_ref, out_ref):
    acc = acc_ref[0] + acc_ref[1]
    den_e = acc[:, 0]
    num = acc[:, 1:7]
    deg = acc[:, 7]
    asum = acc[:, 8:10]
    sl = asum / jnp.maximum(deg, 1.0)[:, None]
    xl = xl_ref[:, :6]
    xr = xr_ref[:, :6]
    msg = xl + xr + sl @ we_ref[...]
    lk = jnp.maximum(msg, 0.2 * msg)
    logit = jnp.sum(lk * att_ref[...], axis=1)
    ex = jnp.exp(logit)
    den = den_e + ex
    numt = num + ex[:, None] * xl
    out0 = numt / (den[:, None] + 1e-16) + bias_ref[...]
    h = jnp.maximum(out0, 0.0)
    h = jnp.maximum(h @ w1_ref[...] + b1_ref[...], 0.0)
    h = jnp.maximum(h @ w2_ref[...] + b2_ref[...], 0.0)
    out_ref[...] = h * 2.0 - 1.0


def _finish(acc2, xl_pad, xr_pad, we, att2, bias2, w1, b12, w2, b22):
    grid = (_N // _BLK,)
    return pl.pallas_call(
        _finish_body,
        grid=grid,
        in_specs=[
            pl.BlockSpec((2, _BLK, _AW), lambda i: (0, i, 0)),
            pl.BlockSpec((_BLK, 8), lambda i: (i, 0)),
            pl.BlockSpec((_BLK, 8), lambda i: (i, 0)),
            pl.BlockSpec((2, _C), lambda i: (0, 0)),
            pl.BlockSpec((1, _C), lambda i: (0, 0)),
            pl.BlockSpec((1, _C), lambda i: (0, 0)),
            pl.BlockSpec((_C, _C), lambda i: (0, 0)),
            pl.BlockSpec((1, _C), lambda i: (0, 0)),
            pl.BlockSpec((_C, _OUT), lambda i: (0, 0)),
            pl.BlockSpec((1, _OUT), lambda i: (0, 0)),
        ],
        out_specs=pl.BlockSpec((_BLK, _OUT), lambda i: (i, 0)),
        out_shape=jax.ShapeDtypeStruct((_N, _OUT), jnp.float32),
    )(acc2, xl_pad, xr_pad, we, att2, bias2, w1, b12, w2, b22)


def kernel(x, edge_index, edge_attr, W_l, b_l, W_r, b_r, W_e, att, bias,
           W1, b1, W2, b2):
    wl8 = jnp.pad(W_l, ((0, 0), (0, 2)))
    bl8 = jnp.pad(b_l, (0, 2)).reshape(1, 8)
    wr8 = jnp.pad(W_r, ((0, 0), (0, 2)))
    br8 = jnp.pad(b_r, (0, 2)).reshape(1, 8)
    xl_pad, xr_pad = _prep(x, wl8, bl8, wr8, br8)

    pad = _EP - _E
    src_p = jnp.concatenate(
        [edge_index[0], jnp.zeros((pad,), jnp.int32)]).reshape(_EP // 128, 128)
    dst_p = jnp.concatenate(
        [edge_index[1], jnp.full((pad,), _N, jnp.int32)]).reshape(_EP // 128, 128)
    a0_p = jnp.pad(edge_attr[:, 0], (0, pad))
    a1_p = jnp.pad(edge_attr[:, 1], (0, pad))
    params = jnp.concatenate(
        [W_e.reshape(-1), att, jnp.zeros((6,), jnp.float32)])
    zeros_acc = jnp.zeros((_ACC_N, _AW), jnp.float32)

    acc2 = _edge_kernel(src_p, dst_p, a0_p, a1_p, xl_pad, xr_pad,
                        zeros_acc, params)

    return _finish(acc2, xl_pad, xr_pad, W_e, att.reshape(1, _C),
                   bias.reshape(1, _C), W1, b1.reshape(1, _C), W2,
                   b2.reshape(1, _OUT))


# speedup vs baseline: 104.6252x; 1.3711x over previous
import functools

import jax
import jax.numpy as jnp
from jax import lax
from jax.experimental import pallas as pl
from jax.experimental.pallas import tpu as pltpu
from jax.experimental.pallas import tpu_sc as plsc

_N = 100000
_E = 6400000
_C = 6
_OUT = 3

_K = 256
_SUB = _K // 128
_NW = 32
_PT = 200704
_EP = _PT * _NW
_NCH = _PT // _K
_ACC_N = 100016
_AW = 16
_ZR = _ACC_N // 16
_WR = _N // 16
_BLK = 2000


def _prep_body(x_ref, wl_ref, bl_ref, wr_ref, br_ref, xl_ref, xr_ref):
    x = x_ref[...]
    xl_ref[...] = x @ wl_ref[...] + bl_ref[...]
    xr_ref[...] = x @ wr_ref[...] + br_ref[...]


def _prep(x, wl8, bl8, wr8, br8):
    grid = (_N // _BLK,)
    return pl.pallas_call(
        _prep_body,
        grid=grid,
        in_specs=[
            pl.BlockSpec((_BLK, _C), lambda i: (i, 0)),
            pl.BlockSpec((_C, 8), lambda i: (0, 0)),
            pl.BlockSpec((1, 8), lambda i: (0, 0)),
            pl.BlockSpec((_C, 8), lambda i: (0, 0)),
            pl.BlockSpec((1, 8), lambda i: (0, 0)),
        ],
        out_specs=[
            pl.BlockSpec((_BLK, 8), lambda i: (i, 0)),
            pl.BlockSpec((_BLK, 8), lambda i: (i, 0)),
        ],
        out_shape=[
            jax.ShapeDtypeStruct((_N, 8), jnp.float32),
            jax.ShapeDtypeStruct((_N, 8), jnp.float32),
        ],
    )(x, wl8, bl8, wr8, br8)


_GRP = _K // 16


def _edge_body(src_hbm, dst_hbm, a0_hbm, a1_hbm, xl_hbm, xr_hbm, zero_hbm,
               par_hbm, out_hbm, acc,
               src_v0, src_v1, src_v2, src_v3,
               dst_v0, dst_v1, dst_v2, dst_v3,
               a0_v0, a0_v1, a0_v2, a0_v3,
               a1_v0, a1_v1, a1_v2, a1_v3,
               xl_v0, xl_v1, xr_v0, xr_v1, s_v0, s_v1, par_v,
               semA0, semA1, semA2, semA3, semB0, semB1,
               semD0, semD1, semD2, semD3):
    cid = lax.axis_index("c")
    sid = lax.axis_index("s")
    wid = sid * 2 + cid

    src_vs = [src_v0, src_v1, src_v2, src_v3]
    dst_vs = [dst_v0, dst_v1, dst_v2, dst_v3]
    a0_vs = [a0_v0, a0_v1, a0_v2, a0_v3]
    a1_vs = [a1_v0, a1_v1, a1_v2, a1_v3]
    xl_vs = [xl_v0, xl_v1]
    xr_vs = [xr_v0, xr_v1]
    s_vs = [s_v0, s_v1]
    semA = [semA0, semA1, semA2, semA3]
    semB = [semB0, semB1]
    semD = [semD0, semD1, semD2, semD3]

    pltpu.sync_copy(par_hbm, par_v)
    pltpu.sync_copy(zero_hbm.at[pl.ds(0, _K)], s_v0)
    pltpu.sync_copy(zero_hbm.at[pl.ds(0, _K)], s_v1)
    pltpu.sync_copy(zero_hbm.at[pl.ds(sid * _ZR, _ZR)],
                    acc.at[pl.ds(sid * _ZR, _ZR)])
    plsc.subcore_barrier()

    lanes = lax.iota(jnp.int32, 16)
    cols = [jnp.full((16,), j, jnp.int32) for j in range(10)]
    p_lo = par_v[pl.ds(0, 16)]
    p_hi = par_v[pl.ds(8, 16)]
    we0 = [p_lo[j] for j in range(6)]
    we1 = [p_lo[6 + j] for j in range(6)]
    att_s = [p_hi[4 + j] for j in range(6)]
    ones16 = jnp.ones((16,), jnp.float32)

    def issue_a(i, slot):
        e_base = wid * _PT + i * _K
        r_base = wid * (_PT // 128) + i * _SUB
        pltpu.async_copy(src_hbm.at[pl.ds(r_base, _SUB)], src_vs[slot],
                         semA[slot])
        pltpu.async_copy(dst_hbm.at[pl.ds(r_base, _SUB)], dst_vs[slot],
                         semA[slot])
        pltpu.async_copy(a0_hbm.at[pl.ds(e_base, _K)], a0_vs[slot],
                         semA[slot])
        pltpu.async_copy(a1_hbm.at[pl.ds(e_base, _K)], a1_vs[slot],
                         semA[slot])

    def drain_a(slot):
        pltpu.make_async_copy(src_hbm.at[pl.ds(0, _SUB)], src_vs[slot],
                              semA[slot]).wait()
        pltpu.make_async_copy(dst_hbm.at[pl.ds(0, _SUB)], dst_vs[slot],
                              semA[slot]).wait()
        pltpu.make_async_copy(a0_hbm.at[pl.ds(0, _K)], a0_vs[slot],
                              semA[slot]).wait()
        pltpu.make_async_copy(a1_hbm.at[pl.ds(0, _K)], a1_vs[slot],
                              semA[slot]).wait()

    def issue_b(slot, g):
        for c in range(_SUB):
            pltpu.async_copy(xl_hbm.at[src_vs[slot].at[c]],
                             xl_vs[g].at[pl.ds(c * 128, 128)], semB[g])
            pltpu.async_copy(xr_hbm.at[dst_vs[slot].at[c]],
                             xr_vs[g].at[pl.ds(c * 128, 128)], semB[g])

    def drain_b(g):
        for c in range(_SUB):
            pltpu.make_async_copy(xl_hbm.at[pl.ds(0, 128)],
                                  xl_vs[g].at[pl.ds(c * 128, 128)],
                                  semB[g]).wait()
            pltpu.make_async_copy(xr_hbm.at[pl.ds(0, 128)],
                                  xr_vs[g].at[pl.ds(c * 128, 128)],
                                  semB[g]).wait()

    def issue_d(slot, g):
        for c in range(_SUB):
            pltpu.async_copy(s_vs[g].at[pl.ds(c * 128, 128)],
                             acc.at[dst_vs[slot].at[c]], semD[slot],
                             add=True)

    def drain_d(slot):
        for c in range(_SUB):
            pltpu.make_async_copy(zero_hbm.at[pl.ds(0, 128)],
                                  s_vs[0].at[pl.ds(c * 128, 128)],
                                  semD[slot]).wait()

    def compute(slot, g):
        xl_v = xl_vs[g]
        xr_v = xr_vs[g]
        s_v = s_vs[g]
        a0_r = a0_vs[slot]
        a1_r = a1_vs[slot]

        def group_body(gi, c_):
            eidx = gi * 16 + lanes
            a0 = a0_r[pl.ds(gi * 16, 16)]
            a1 = a1_r[pl.ds(gi * 16, 16)]
            xl = [plsc.load_gather(xl_v, [eidx, cols[j]]) for j in range(6)]
            xr = [plsc.load_gather(xr_v, [eidx, cols[j]]) for j in range(6)]
            logit = jnp.zeros((16,), jnp.float32)
            for j in range(6):
                m = xl[j] + xr[j] + a0 * we0[j] + a1 * we1[j]
                lk = jnp.maximum(m, m * 0.2)
                logit = logit + lk * att_s[j]
            ex = jnp.exp(logit)
            plsc.store_scatter(s_v, [eidx, cols[0]], ex)
            for j in range(6):
                plsc.store_scatter(s_v, [eidx, cols[1 + j]], ex * xl[j])
            plsc.store_scatter(s_v, [eidx, cols[7]], ones16)
            plsc.store_scatter(s_v, [eidx, cols[8]], a0)
            plsc.store_scatter(s_v, [eidx, cols[9]], a1)
            return c_

        lax.fori_loop(0, _GRP, group_body, 0)

    issue_a(0, 0)
    issue_a(1, 1)
    drain_a(0)
    issue_b(0, 0)

    n_quads = _NCH // 4

    def quad_body(mq, c_):
        for k in range(4):
            g = k % 2
            i = mq * 4 + k

            drain_b(g)
            if k < 2:
                @pl.when(mq > 0)
                def _():
                    drain_d((k + 2) % 4)
            else:
                drain_d((k + 2) % 4)
            if k < 2:
                issue_a(i + 2, (k + 2) % 4)
            else:
                @pl.when(mq < n_quads - 1)
                def _():
                    issue_a(i + 2, (k + 2) % 4)
            if k == 3:
                @pl.when(mq < n_quads - 1)
                def _():
                    drain_a(0)
                    issue_b(0, 0)
            else:
                drain_a((k + 1) % 4)
                issue_b((k + 1) % 4, (k + 1) % 2)
            compute(k, g)
            issue_d(k, g)
        return c_

    lax.fori_loop(0, n_quads, quad_body, 0)
    drain_d(2)
    drain_d(3)
    plsc.subcore_barrier()
    pltpu.sync_copy(acc.at[pl.ds(sid * _WR, _WR)],
                    out_hbm.at[cid, pl.ds(sid * _WR, _WR)])


_edge_kernel = functools.partial(
    pl.kernel,
    out_type=jax.ShapeDtypeStruct((2, _N, _AW), jnp.float32),
    mesh=plsc.VectorSubcoreMesh(core_axis_name="c", subcore_axis_name="s"),
    compiler_params=pltpu.CompilerParams(use_tc_tiling_on_sc=False,
                                         needs_layout_passes=False),
    scratch_types=(
        [pltpu.VMEM_SHARED((_ACC_N, _AW), jnp.float32)]
        + [pltpu.VMEM((_SUB, 128), jnp.int32) for _ in range(8)]
        + [pltpu.VMEM((_K,), jnp.float32) for _ in range(8)]
        + [pltpu.VMEM((_K, 8), jnp.float32) for _ in range(4)]
        + [pltpu.VMEM((_K, _AW), jnp.float32) for _ in range(2)]
        + [pltpu.VMEM((24,), jnp.float32)]
        + [pltpu.SemaphoreType.DMA for _ in range(10)]
    ),
)(_edge_body)


def _finish_body(acc_ref, xl_ref, xr_ref, we_ref, att_ref, bias_ref,
                 w1_ref, b1_ref, w2_ref, b2_ref, out_ref):
    acc = acc_ref[0] + acc_ref[1]
    den_e = acc[:, 0]
    num = acc[:, 1:7]
    deg = acc[:, 7]
    asum = acc[:, 8:10]
    sl = asum / jnp.maximum(deg, 1.0)[:, None]
    xl = xl_ref[:, :6]
    xr = xr_ref[:, :6]
    msg = xl + xr + sl @ we_ref[...]
    lk = jnp.maximum(msg, 0.2 * msg)
    logit = jnp.sum(lk * att_ref[...], axis=1)
    ex = jnp.exp(logit)
    den = den_e + ex
    numt = num + ex[:, None] * xl
    out0 = numt / (den[:, None] + 1e-16) + bias_ref[...]
    h = jnp.maximum(out0, 0.0)
    h = jnp.maximum(h @ w1_ref[...] + b1_ref[...], 0.0)
    h = jnp.maximum(h @ w2_ref[...] + b2_ref[...], 0.0)
    out_ref[...] = h * 2.0 - 1.0


def _finish(acc2, xl_pad, xr_pad, we, att2, bias2, w1, b12, w2, b22):
    grid = (_N // _BLK,)
    return pl.pallas_call(
        _finish_body,
        grid=grid,
        in_specs=[
            pl.BlockSpec((2, _BLK, _AW), lambda i: (0, i, 0)),
            pl.BlockSpec((_BLK, 8), lambda i: (i, 0)),
            pl.BlockSpec((_BLK, 8), lambda i: (i, 0)),
            pl.BlockSpec((2, _C), lambda i: (0, 0)),
            pl.BlockSpec((1, _C), lambda i: (0, 0)),
            pl.BlockSpec((1, _C), lambda i: (0, 0)),
            pl.BlockSpec((_C, _C), lambda i: (0, 0)),
            pl.BlockSpec((1, _C), lambda i: (0, 0)),
            pl.BlockSpec((_C, _OUT), lambda i: (0, 0)),
            pl.BlockSpec((1, _OUT), lambda i: (0, 0)),
        ],
        out_specs=pl.BlockSpec((_BLK, _OUT), lambda i: (i, 0)),
        out_shape=jax.ShapeDtypeStruct((_N, _OUT), jnp.float32),
    )(acc2, xl_pad, xr_pad, we, att2, bias2, w1, b12, w2, b22)


def kernel(x, edge_index, edge_attr, W_l, b_l, W_r, b_r, W_e, att, bias,
           W1, b1, W2, b2):
    wl8 = jnp.pad(W_l, ((0, 0), (0, 2)))
    bl8 = jnp.pad(b_l, (0, 2)).reshape(1, 8)
    wr8 = jnp.pad(W_r, ((0, 0), (0, 2)))
    br8 = jnp.pad(b_r, (0, 2)).reshape(1, 8)
    xl_pad, xr_pad = _prep(x, wl8, bl8, wr8, br8)

    pad = _EP - _E
    src_p = jnp.concatenate(
        [edge_index[0], jnp.zeros((pad,), jnp.int32)]).reshape(_EP // 128, 128)
    dst_p = jnp.concatenate(
        [edge_index[1], jnp.full((pad,), _N, jnp.int32)]).reshape(_EP // 128, 128)
    a0_p = jnp.pad(edge_attr[:, 0], (0, pad))
    a1_p = jnp.pad(edge_attr[:, 1], (0, pad))
    params = jnp.concatenate(
        [W_e.reshape(-1), att, jnp.zeros((6,), jnp.float32)])
    zeros_acc = jnp.zeros((_ACC_N, _AW), jnp.float32)

    acc2 = _edge_kernel(src_p, dst_p, a0_p, a1_p, xl_pad, xr_pad,
                        zeros_acc, params)

    return _finish(acc2, xl_pad, xr_pad, W_e, att.reshape(1, _C),
                   bias.reshape(1, _C), W1, b1.reshape(1, _C), W2,
                   b2.reshape(1, _OUT))


# speedup vs baseline: 106.8951x; 1.0217x over previous
import functools

import jax
import jax.numpy as jnp
from jax import lax
from jax.experimental import pallas as pl
from jax.experimental.pallas import tpu as pltpu
from jax.experimental.pallas import tpu_sc as plsc

_N = 100000
_E = 6400000
_C = 6
_OUT = 3

_K = 256
_SUB = _K // 128
_NW = 32
_PT = 200704
_EP = _PT * _NW
_NCH = _PT // _K
_ACC_N = 100016
_AW = 16
_ZR = _ACC_N // 16
_WR = _N // 16
_BLK = 5000


def _prep_body(x_ref, wl_ref, bl_ref, wr_ref, br_ref, xl_ref, xr_ref):
    x = x_ref[...]
    xl_ref[...] = x @ wl_ref[...] + bl_ref[...]
    xr_ref[...] = x @ wr_ref[...] + br_ref[...]


def _prep(x, wl8, bl8, wr8, br8):
    grid = (_N // _BLK,)
    return pl.pallas_call(
        _prep_body,
        grid=grid,
        in_specs=[
            pl.BlockSpec((_BLK, _C), lambda i: (i, 0)),
            pl.BlockSpec((_C, 8), lambda i: (0, 0)),
            pl.BlockSpec((1, 8), lambda i: (0, 0)),
            pl.BlockSpec((_C, 8), lambda i: (0, 0)),
            pl.BlockSpec((1, 8), lambda i: (0, 0)),
        ],
        out_specs=[
            pl.BlockSpec((_BLK, 8), lambda i: (i, 0)),
            pl.BlockSpec((_BLK, 8), lambda i: (i, 0)),
        ],
        out_shape=[
            jax.ShapeDtypeStruct((_N, 8), jnp.float32),
            jax.ShapeDtypeStruct((_N, 8), jnp.float32),
        ],
    )(x, wl8, bl8, wr8, br8)


_GRP = _K // 16


def _edge_body(src_hbm, dst_hbm, a0_hbm, a1_hbm, xl_hbm, xr_hbm, zero_hbm,
               par_hbm, out_hbm, acc,
               src_v0, src_v1, src_v2, src_v3,
               dst_v0, dst_v1, dst_v2, dst_v3,
               a0_v0, a0_v1, a0_v2, a0_v3,
               a1_v0, a1_v1, a1_v2, a1_v3,
               xl_v0, xl_v1, xr_v0, xr_v1, s_v0, s_v1, par_v,
               semA0, semA1, semA2, semA3, semB0, semB1,
               semD0, semD1, semD2, semD3):
    cid = lax.axis_index("c")
    sid = lax.axis_index("s")
    wid = sid * 2 + cid

    src_vs = [src_v0, src_v1, src_v2, src_v3]
    dst_vs = [dst_v0, dst_v1, dst_v2, dst_v3]
    a0_vs = [a0_v0, a0_v1, a0_v2, a0_v3]
    a1_vs = [a1_v0, a1_v1, a1_v2, a1_v3]
    xl_vs = [xl_v0, xl_v1]
    xr_vs = [xr_v0, xr_v1]
    s_vs = [s_v0, s_v1]
    semA = [semA0, semA1, semA2, semA3]
    semB = [semB0, semB1]
    semD = [semD0, semD1, semD2, semD3]

    pltpu.sync_copy(par_hbm, par_v)
    pltpu.sync_copy(zero_hbm.at[pl.ds(0, _K)], s_v0)
    pltpu.sync_copy(zero_hbm.at[pl.ds(0, _K)], s_v1)
    pltpu.sync_copy(zero_hbm.at[pl.ds(sid * _ZR, _ZR)],
                    acc.at[pl.ds(sid * _ZR, _ZR)])
    plsc.subcore_barrier()

    lanes = lax.iota(jnp.int32, 16)
    cols = [jnp.full((16,), j, jnp.int32) for j in range(10)]
    p_lo = par_v[pl.ds(0, 16)]
    p_hi = par_v[pl.ds(8, 16)]
    we0 = [p_lo[j] for j in range(6)]
    we1 = [p_lo[6 + j] for j in range(6)]
    att_s = [p_hi[4 + j] for j in range(6)]
    ones16 = jnp.ones((16,), jnp.float32)

    def issue_a(i, slot):
        e_base = wid * _PT + i * _K
        r_base = wid * (_PT // 128) + i * _SUB
        pltpu.async_copy(src_hbm.at[pl.ds(r_base, _SUB)], src_vs[slot],
                         semA[slot])
        pltpu.async_copy(dst_hbm.at[pl.ds(r_base, _SUB)], dst_vs[slot],
                         semA[slot])
        pltpu.async_copy(a0_hbm.at[pl.ds(e_base, _K)], a0_vs[slot],
                         semA[slot])
        pltpu.async_copy(a1_hbm.at[pl.ds(e_base, _K)], a1_vs[slot],
                         semA[slot])

    def drain_a(slot):
        pltpu.make_async_copy(src_hbm.at[pl.ds(0, _SUB)], src_vs[slot],
                              semA[slot]).wait()
        pltpu.make_async_copy(dst_hbm.at[pl.ds(0, _SUB)], dst_vs[slot],
                              semA[slot]).wait()
        pltpu.make_async_copy(a0_hbm.at[pl.ds(0, _K)], a0_vs[slot],
                              semA[slot]).wait()
        pltpu.make_async_copy(a1_hbm.at[pl.ds(0, _K)], a1_vs[slot],
                              semA[slot]).wait()

    def issue_b(slot, g):
        for c in range(_SUB):
            pltpu.async_copy(xl_hbm.at[src_vs[slot].at[c]],
                             xl_vs[g].at[pl.ds(c * 128, 128)], semB[g])
            pltpu.async_copy(xr_hbm.at[dst_vs[slot].at[c]],
                             xr_vs[g].at[pl.ds(c * 128, 128)], semB[g])

    def drain_b(g):
        for c in range(_SUB):
            pltpu.make_async_copy(xl_hbm.at[pl.ds(0, 128)],
                                  xl_vs[g].at[pl.ds(c * 128, 128)],
                                  semB[g]).wait()
            pltpu.make_async_copy(xr_hbm.at[pl.ds(0, 128)],
                                  xr_vs[g].at[pl.ds(c * 128, 128)],
                                  semB[g]).wait()

    def issue_d(slot, g):
        for c in range(_SUB):
            pltpu.async_copy(s_vs[g].at[pl.ds(c * 128, 128)],
                             acc.at[dst_vs[slot].at[c]], semD[slot],
                             add=True)

    def drain_d(slot):
        for c in range(_SUB):
            pltpu.make_async_copy(zero_hbm.at[pl.ds(0, 128)],
                                  s_vs[0].at[pl.ds(c * 128, 128)],
                                  semD[slot]).wait()

    def compute(slot, g):
        xl_v = xl_vs[g]
        xr_v = xr_vs[g]
        s_v = s_vs[g]
        a0_r = a0_vs[slot]
        a1_r = a1_vs[slot]

        def group_body(gi, c_):
            eidx = gi * 16 + lanes
            a0 = a0_r[pl.ds(gi * 16, 16)]
            a1 = a1_r[pl.ds(gi * 16, 16)]
            xl = [plsc.load_gather(xl_v, [eidx, cols[j]]) for j in range(6)]
            xr = [plsc.load_gather(xr_v, [eidx, cols[j]]) for j in range(6)]
            logit = jnp.zeros((16,), jnp.float32)
            for j in range(6):
                m = xl[j] + xr[j] + a0 * we0[j] + a1 * we1[j]
                lk = jnp.maximum(m, m * 0.2)
                logit = logit + lk * att_s[j]
            ex = jnp.exp(logit)
            plsc.store_scatter(s_v, [eidx, cols[0]], ex)
            for j in range(6):
                plsc.store_scatter(s_v, [eidx, cols[1 + j]], ex * xl[j])
            plsc.store_scatter(s_v, [eidx, cols[7]], ones16)
            plsc.store_scatter(s_v, [eidx, cols[8]], a0)
            plsc.store_scatter(s_v, [eidx, cols[9]], a1)
            return c_

        lax.fori_loop(0, _GRP, group_body, 0, unroll=2)

    issue_a(0, 0)
    issue_a(1, 1)
    drain_a(0)
    issue_b(0, 0)

    n_quads = _NCH // 4

    def quad_body(mq, c_):
        for k in range(4):
            g = k % 2
            i = mq * 4 + k

            drain_b(g)
            if k < 2:
                @pl.when(mq > 0)
                def _():
                    drain_d((k + 2) % 4)
            else:
                drain_d((k + 2) % 4)
            if k < 2:
                issue_a(i + 2, (k + 2) % 4)
            else:
                @pl.when(mq < n_quads - 1)
                def _():
                    issue_a(i + 2, (k + 2) % 4)
            if k == 3:
                @pl.when(mq < n_quads - 1)
                def _():
                    drain_a(0)
                    issue_b(0, 0)
            else:
                drain_a((k + 1) % 4)
                issue_b((k + 1) % 4, (k + 1) % 2)
            compute(k, g)
            issue_d(k, g)
        return c_

    lax.fori_loop(0, n_quads, quad_body, 0)
    drain_d(2)
    drain_d(3)
    plsc.subcore_barrier()
    pltpu.sync_copy(acc.at[pl.ds(sid * _WR, _WR)],
                    out_hbm.at[cid, pl.ds(sid * _WR, _WR)])


_edge_kernel = functools.partial(
    pl.kernel,
    out_type=jax.ShapeDtypeStruct((2, _N, _AW), jnp.float32),
    mesh=plsc.VectorSubcoreMesh(core_axis_name="c", subcore_axis_name="s"),
    compiler_params=pltpu.CompilerParams(use_tc_tiling_on_sc=False,
                                         needs_layout_passes=False),
    scratch_types=(
        [pltpu.VMEM_SHARED((_ACC_N, _AW), jnp.float32)]
        + [pltpu.VMEM((_SUB, 128), jnp.int32) for _ in range(8)]
        + [pltpu.VMEM((_K,), jnp.float32) for _ in range(8)]
        + [pltpu.VMEM((_K, 8), jnp.float32) for _ in range(4)]
        + [pltpu.VMEM((_K, _AW), jnp.float32) for _ in range(2)]
        + [pltpu.VMEM((24,), jnp.float32)]
        + [pltpu.SemaphoreType.DMA for _ in range(10)]
    ),
)(_edge_body)


def _finish_body(acc_ref, xl_ref, xr_ref, we_ref, att_ref, bias_ref,
                 w1_ref, b1_ref, w2_ref, b2_ref, out_ref):
    acc = acc_ref[0] + acc_ref[1]
    den_e = acc[:, 0]
    num = acc[:, 1:7]
    deg = acc[:, 7]
    asum = acc[:, 8:10]
    sl = asum / jnp.maximum(deg, 1.0)[:, None]
    xl = xl_ref[:, :6]
    xr = xr_ref[:, :6]
    msg = xl + xr + sl @ we_ref[...]
    lk = jnp.maximum(msg, 0.2 * msg)
    logit = jnp.sum(lk * att_ref[...], axis=1)
    ex = jnp.exp(logit)
    den = den_e + ex
    numt = num + ex[:, None] * xl
    out0 = numt / (den[:, None] + 1e-16) + bias_ref[...]
    h = jnp.maximum(out0, 0.0)
    h = jnp.maximum(h @ w1_ref[...] + b1_ref[...], 0.0)
    h = jnp.maximum(h @ w2_ref[...] + b2_ref[...], 0.0)
    out_ref[...] = h * 2.0 - 1.0


def _finish(acc2, xl_pad, xr_pad, we, att2, bias2, w1, b12, w2, b22):
    grid = (_N // _BLK,)
    return pl.pallas_call(
        _finish_body,
        grid=grid,
        in_specs=[
            pl.BlockSpec((2, _BLK, _AW), lambda i: (0, i, 0)),
            pl.BlockSpec((_BLK, 8), lambda i: (i, 0)),
            pl.BlockSpec((_BLK, 8), lambda i: (i, 0)),
            pl.BlockSpec((2, _C), lambda i: (0, 0)),
            pl.BlockSpec((1, _C), lambda i: (0, 0)),
            pl.BlockSpec((1, _C), lambda i: (0, 0)),
            pl.BlockSpec((_C, _C), lambda i: (0, 0)),
            pl.BlockSpec((1, _C), lambda i: (0, 0)),
            pl.BlockSpec((_C, _OUT), lambda i: (0, 0)),
            pl.BlockSpec((1, _OUT), lambda i: (0, 0)),
        ],
        out_specs=pl.BlockSpec((_BLK, _OUT), lambda i: (i, 0)),
        out_shape=jax.ShapeDtypeStruct((_N, _OUT), jnp.float32),
    )(acc2, xl_pad, xr_pad, we, att2, bias2, w1, b12, w2, b22)


def kernel(x, edge_index, edge_attr, W_l, b_l, W_r, b_r, W_e, att, bias,
           W1, b1, W2, b2):
    wl8 = jnp.pad(W_l, ((0, 0), (0, 2)))
    bl8 = jnp.pad(b_l, (0, 2)).reshape(1, 8)
    wr8 = jnp.pad(W_r, ((0, 0), (0, 2)))
    br8 = jnp.pad(b_r, (0, 2)).reshape(1, 8)
    xl_pad, xr_pad = _prep(x, wl8, bl8, wr8, br8)

    pad = _EP - _E
    src_p = jnp.concatenate(
        [edge_index[0], jnp.zeros((pad,), jnp.int32)]).reshape(_EP // 128, 128)
    dst_p = jnp.concatenate(
        [edge_index[1], jnp.full((pad,), _N, jnp.int32)]).reshape(_EP // 128, 128)
    a0_p = jnp.pad(edge_attr[:, 0], (0, pad))
    a1_p = jnp.pad(edge_attr[:, 1], (0, pad))
    params = jnp.concatenate(
        [W_e.reshape(-1), att, jnp.zeros((6,), jnp.float32)])
    zeros_acc = jnp.zeros((_ACC_N, _AW), jnp.float32)

    acc2 = _edge_kernel(src_p, dst_p, a0_p, a1_p, xl_pad, xr_pad,
                        zeros_acc, params)

    return _finish(acc2, xl_pad, xr_pad, W_e, att.reshape(1, _C),
                   bias.reshape(1, _C), W1, b1.reshape(1, _C), W2,
                   b2.reshape(1, _OUT))
